# baseline (speedup 1.0000x reference)
import functools

import jax
import jax.numpy as jnp
from jax import lax
from jax.experimental import pallas as pl
from jax.experimental.pallas import tpu as pltpu, tpu_sc as plsc

VOCAB = 1000000
D = 64
DP = 128
B = 16384
C = 20
K = 20

NC = 2
NS = 16
NW = NC * NS

NJ = VOCAB // DP
JT = NJ * DP
NTAIL = VOCAB - JT
J_PER_W = NJ // NW
J_EXTRA = NJ - J_PER_W * NW

ROWS_W = B // NW
R = 16
NCHUNK = ROWS_W // R
IDX_BLK = 80
NBLK = R * C // IDX_BLK


def _relayout_body(inT_hbm, outT_hbm, tail_hbm, comb_hbm,
                   in_slab, out_slab, dst_v, sem):
    wid = lax.axis_index("s") * NC + lax.axis_index("c")
    lane = lax.iota(jnp.int32, 16)
    rows_g = [g * 16 + lane for g in range(4)]

    base = wid * J_PER_W + jnp.minimum(wid, J_EXTRA)
    cnt = J_PER_W + jnp.where(wid < J_EXTRA, 1, 0)

    def blk_body(j, carry):
        jj = base + j
        pltpu.sync_copy(inT_hbm.at[:, pl.ds(jj * DP, DP)], in_slab)
        pltpu.sync_copy(outT_hbm.at[:, pl.ds(jj * DP, DP)], out_slab)

        def col_body(c, ccarry):
            col = jnp.full((16,), c, jnp.int32)
            for g in range(4):
                dst_v[c, pl.ds(g * 16, 16)] = plsc.load_gather(
                    in_slab, [rows_g[g], col])
                dst_v[c, pl.ds(D + g * 16, 16)] = plsc.load_gather(
                    out_slab, [rows_g[g], col])
            return ccarry

        lax.fori_loop(0, DP, col_body, 0)
        pltpu.sync_copy(dst_v, comb_hbm.at[pl.ds(jj * DP, DP), :])
        return carry

    lax.fori_loop(0, cnt, blk_body, 0)

    @pl.when(wid == NW - 1)
    def _():
        pltpu.sync_copy(tail_hbm, in_slab)
        pltpu.sync_copy(in_slab, comb_hbm.at[pl.ds(JT, NTAIL), :])


def _cbow_body(tgt_hbm, ctxidx_hbm, negidx_hbm, comb_hbm,
               out_hbm, idx_ctx_v, idx_neg_v, idx_tgt_v, ctx_rows_v,
               orow_v, out_v, sem):
    wid = lax.axis_index("s") * NC + lax.axis_index("c")
    lane = lax.iota(jnp.int32, 16)

    def chunk_body(ch, carry):
        row0 = wid * ROWS_W + ch * R
        off = row0 * C

        pltpu.sync_copy(ctxidx_hbm.at[pl.ds(off, R * C)], idx_ctx_v)
        pltpu.sync_copy(negidx_hbm.at[pl.ds(off, R * K)], idx_neg_v)
        pltpu.sync_copy(tgt_hbm.at[pl.ds(row0, R)], idx_tgt_v)

        copies = []
        for j in range(NBLK):
            copies.append(pltpu.async_copy(
                comb_hbm.at[idx_ctx_v.at[pl.ds(j * IDX_BLK, IDX_BLK)]],
                ctx_rows_v.at[pl.ds(j * IDX_BLK, IDX_BLK)], sem))
            copies.append(pltpu.async_copy(
                comb_hbm.at[idx_neg_v.at[pl.ds(j * IDX_BLK, IDX_BLK)]],
                orow_v.at[pl.ds(j * IDX_BLK, IDX_BLK)], sem))
        copies.append(pltpu.async_copy(
            comb_hbm.at[idx_tgt_v], orow_v.at[pl.ds(R * K, R)], sem))
        for cp in copies:
            cp.wait()

        def row_body(r, rcarry):
            base = r * C
            acc = [ctx_rows_v[base, pl.ds(j * 16, 16)] for j in range(4)]
            for c in range(1, C):
                for j in range(4):
                    acc[j] = acc[j] + ctx_rows_v[base + c, pl.ds(j * 16, 16)]
            inv_c = jnp.float32(1.0 / C)
            ctxv = [acc[j] * inv_c for j in range(4)]

            def dot(row):
                v = ctxv[0] * orow_v[row, pl.ds(D, 16)]
                for j in range(1, 4):
                    v = v + ctxv[j] * orow_v[row, pl.ds(D + j * 16, 16)]
                return jnp.sum(v)

            acc1 = jnp.zeros((16,), jnp.float32)
            acc2 = jnp.where(lane == 4, dot(R * K + r),
                             jnp.zeros((16,), jnp.float32))
            for k in range(K):
                s = dot(base + k)
                if k < 16:
                    acc1 = jnp.where(lane == k, s, acc1)
                else:
                    acc2 = jnp.where(lane == k - 16, s, acc2)
            out_v[r, pl.ds(0, 16)] = acc1
            out_v[r, pl.ds(16, 16)] = acc2
            return rcarry

        lax.fori_loop(0, R, row_body, 0)
        pltpu.sync_copy(out_v, out_hbm.at[pl.ds(row0, R)])
        return carry

    lax.fori_loop(0, NCHUNK, chunk_body, 0)


@functools.partial(jax.jit, static_argnums=())
def _cbow_sc(target_ids, ctx_idx, neg_idx, inT, outT, tail_comb):
    mesh = plsc.VectorSubcoreMesh(core_axis_name="c", subcore_axis_name="s")
    params = pltpu.CompilerParams(needs_layout_passes=False)
    k1 = pl.kernel(
        _relayout_body,
        mesh=mesh,
        compiler_params=params,
        out_type=jax.ShapeDtypeStruct((VOCAB, DP), jnp.float32),
        scratch_types=[
            pltpu.VMEM((D, DP), jnp.float32),
            pltpu.VMEM((D, DP), jnp.float32),
            pltpu.VMEM((DP, DP), jnp.float32),
            pltpu.SemaphoreType.DMA,
        ],
    )
    comb = k1(inT, outT, tail_comb)
    k2 = pl.kernel(
        _cbow_body,
        mesh=mesh,
        compiler_params=params,
        out_type=jax.ShapeDtypeStruct((B, 32), jnp.float32),
        scratch_types=[
            pltpu.VMEM((R * C,), jnp.int32),
            pltpu.VMEM((R * K,), jnp.int32),
            pltpu.VMEM((R,), jnp.int32),
            pltpu.VMEM((R * C, DP), jnp.float32),
            pltpu.VMEM((R * K + R, DP), jnp.float32),
            pltpu.VMEM((R, 32), jnp.float32),
            pltpu.SemaphoreType.DMA,
        ],
    )
    return k2(target_ids, ctx_idx, neg_idx, comb)


def kernel(target_ids, context_ids, negative_ids, input_emb, output_emb):
    ctx_idx = context_ids.astype(jnp.int32).reshape(B * C)
    neg_idx = negative_ids.astype(jnp.int32).reshape(B * K)
    tail_comb = jnp.concatenate(
        [input_emb[JT:], output_emb[JT:]], axis=1)
    out = _cbow_sc(target_ids.astype(jnp.int32), ctx_idx, neg_idx,
                   input_emb.T, output_emb.T, tail_comb)
    return (out[:, K], out[:, :K])


# speedup vs baseline: 1.1766x; 1.1766x over previous
import functools

import jax
import jax.numpy as jnp
from jax import lax
from jax.experimental import pallas as pl
from jax.experimental.pallas import tpu as pltpu, tpu_sc as plsc

VOCAB = 1000000
D = 64
DP = 128
B = 16384
C = 20
K = 20

NC = 2
NS = 16
NW = NC * NS

NJ = VOCAB // DP
JT = NJ * DP
NTAIL = VOCAB - JT
J_PER_W = NJ // NW
J_EXTRA = NJ - J_PER_W * NW
PAIRS = J_PER_W // 2

ROWS_W = B // NW
R = 16
NCHUNK = ROWS_W // R
IDX_BLK = 80
NBLK = R * C // IDX_BLK


def _relayout_body(inT_hbm, outT_hbm, tail_hbm, comb_hbm,
                   in_s0, out_s0, dst0, in_s1, out_s1, dst1,
                   semr0, semr1, semw0, semw1):
    wid = lax.axis_index("s") * NC + lax.axis_index("c")
    lane = lax.iota(jnp.int32, 16)
    rows_g = [g * 16 + lane for g in range(4)]

    base = wid * J_PER_W + jnp.minimum(wid, J_EXTRA)
    cnt = J_PER_W + jnp.where(wid < J_EXTRA, 1, 0)
    last = base + cnt - 1

    def fetch(jj, in_s, out_s, semr):
        jc = jnp.minimum(jj, last)
        pltpu.async_copy(inT_hbm.at[:, pl.ds(jc * DP, DP)], in_s, semr)
        pltpu.async_copy(outT_hbm.at[:, pl.ds(jc * DP, DP)], out_s, semr)

    def wait_fetch(in_s, out_s, semr):
        pltpu.make_async_copy(inT_hbm.at[:, pl.ds(0, DP)], in_s, semr).wait()
        pltpu.make_async_copy(outT_hbm.at[:, pl.ds(0, DP)], out_s, semr).wait()

    def wait_write(dst, semw):
        pltpu.make_async_copy(comb_hbm.at[pl.ds(0, DP), :], dst, semw).wait()

    def transpose(jj, in_s, out_s, dst, semw):
        def col_grp(c0, carry):
            for ci in range(16):
                col = c0 * 16 + ci + jnp.zeros((16,), jnp.int32)
                for g in range(4):
                    dst[c0 * 16 + ci, pl.ds(g * 16, 16)] = plsc.load_gather(
                        in_s, [rows_g[g], col])
                    dst[c0 * 16 + ci, pl.ds(D + g * 16, 16)] = (
                        plsc.load_gather(out_s, [rows_g[g], col]))
            return carry

        lax.fori_loop(0, DP // 16, col_grp, 0)
        pltpu.async_copy(dst, comb_hbm.at[pl.ds(jj * DP, DP), :], semw)

    fetch(base, in_s0, out_s0, semr0)

    def pair_body(t, carry):
        j0 = base + 2 * t
        wait_fetch(in_s0, out_s0, semr0)
        fetch(j0 + 1, in_s1, out_s1, semr1)

        @pl.when(t > 0)
        def _():
            wait_write(dst0, semw0)
            wait_write(dst1, semw1)

        transpose(j0, in_s0, out_s0, dst0, semw0)
        wait_fetch(in_s1, out_s1, semr1)
        fetch(j0 + 2, in_s0, out_s0, semr0)
        transpose(j0 + 1, in_s1, out_s1, dst1, semw1)
        return carry

    lax.fori_loop(0, PAIRS, pair_body, 0)

    wait_fetch(in_s0, out_s0, semr0)

    @pl.when(cnt % 2 == 1)
    def _():
        wait_write(dst0, semw0)
        transpose(last, in_s0, out_s0, dst0, semw0)

    wait_write(dst0, semw0)
    wait_write(dst1, semw1)

    @pl.when(wid == NW - 1)
    def _():
        pltpu.sync_copy(tail_hbm, dst0.at[pl.ds(0, NTAIL)])
        pltpu.sync_copy(dst0.at[pl.ds(0, NTAIL)],
                        comb_hbm.at[pl.ds(JT, NTAIL), :])


def _cbow_body(tgt_hbm, ctxidx_hbm, negidx_hbm, comb_hbm,
               out_hbm, idx_ctx_v, idx_neg_v, idx_tgt_v, ctx_rows_v,
               orow_v, out_v, sem):
    wid = lax.axis_index("s") * NC + lax.axis_index("c")
    lane = lax.iota(jnp.int32, 16)

    def chunk_body(ch, carry):
        row0 = wid * ROWS_W + ch * R
        off = row0 * C

        pltpu.sync_copy(ctxidx_hbm.at[pl.ds(off, R * C)], idx_ctx_v)
        pltpu.sync_copy(negidx_hbm.at[pl.ds(off, R * K)], idx_neg_v)
        pltpu.sync_copy(tgt_hbm.at[pl.ds(row0, R)], idx_tgt_v)

        copies = []
        for j in range(NBLK):
            copies.append(pltpu.async_copy(
                comb_hbm.at[idx_ctx_v.at[pl.ds(j * IDX_BLK, IDX_BLK)]],
                ctx_rows_v.at[pl.ds(j * IDX_BLK, IDX_BLK)], sem))
            copies.append(pltpu.async_copy(
                comb_hbm.at[idx_neg_v.at[pl.ds(j * IDX_BLK, IDX_BLK)]],
                orow_v.at[pl.ds(j * IDX_BLK, IDX_BLK)], sem))
        copies.append(pltpu.async_copy(
            comb_hbm.at[idx_tgt_v], orow_v.at[pl.ds(R * K, R)], sem))
        for cp in copies:
            cp.wait()

        def row_body(r, rcarry):
            base = r * C
            acc = [ctx_rows_v[base, pl.ds(j * 16, 16)] for j in range(4)]
            for c in range(1, C):
                for j in range(4):
                    acc[j] = acc[j] + ctx_rows_v[base + c, pl.ds(j * 16, 16)]
            inv_c = jnp.float32(1.0 / C)
            ctxv = [acc[j] * inv_c for j in range(4)]

            def dot(row):
                v = ctxv[0] * orow_v[row, pl.ds(D, 16)]
                for j in range(1, 4):
                    v = v + ctxv[j] * orow_v[row, pl.ds(D + j * 16, 16)]
                return jnp.sum(v)

            acc1 = jnp.zeros((16,), jnp.float32)
            acc2 = jnp.where(lane == 4, dot(R * K + r),
                             jnp.zeros((16,), jnp.float32))
            for k in range(K):
                s = dot(base + k)
                if k < 16:
                    acc1 = jnp.where(lane == k, s, acc1)
                else:
                    acc2 = jnp.where(lane == k - 16, s, acc2)
            out_v[r, pl.ds(0, 16)] = acc1
            out_v[r, pl.ds(16, 16)] = acc2
            return rcarry

        lax.fori_loop(0, R, row_body, 0)
        pltpu.sync_copy(out_v, out_hbm.at[pl.ds(row0, R)])
        return carry

    lax.fori_loop(0, NCHUNK, chunk_body, 0)


@functools.partial(jax.jit, static_argnums=())
def _cbow_sc(target_ids, ctx_idx, neg_idx, inT, outT, tail_comb):
    mesh = plsc.VectorSubcoreMesh(core_axis_name="c", subcore_axis_name="s")
    params = pltpu.CompilerParams(needs_layout_passes=False)
    k1 = pl.kernel(
        _relayout_body,
        mesh=mesh,
        compiler_params=params,
        out_type=jax.ShapeDtypeStruct((VOCAB, DP), jnp.float32),
        scratch_types=[
            pltpu.VMEM((D, DP), jnp.float32),
            pltpu.VMEM((D, DP), jnp.float32),
            pltpu.VMEM((DP, DP), jnp.float32),
            pltpu.VMEM((D, DP), jnp.float32),
            pltpu.VMEM((D, DP), jnp.float32),
            pltpu.VMEM((DP, DP), jnp.float32),
            pltpu.SemaphoreType.DMA,
            pltpu.SemaphoreType.DMA,
            pltpu.SemaphoreType.DMA,
            pltpu.SemaphoreType.DMA,
        ],
    )
    comb = k1(inT, outT, tail_comb)
    k2 = pl.kernel(
        _cbow_body,
        mesh=mesh,
        compiler_params=params,
        out_type=jax.ShapeDtypeStruct((B, 32), jnp.float32),
        scratch_types=[
            pltpu.VMEM((R * C,), jnp.int32),
            pltpu.VMEM((R * K,), jnp.int32),
            pltpu.VMEM((R,), jnp.int32),
            pltpu.VMEM((R * C, DP), jnp.float32),
            pltpu.VMEM((R * K + R, DP), jnp.float32),
            pltpu.VMEM((R, 32), jnp.float32),
            pltpu.SemaphoreType.DMA,
        ],
    )
    return k2(target_ids, ctx_idx, neg_idx, comb)


def kernel(target_ids, context_ids, negative_ids, input_emb, output_emb):
    ctx_idx = context_ids.astype(jnp.int32).reshape(B * C)
    neg_idx = negative_ids.astype(jnp.int32).reshape(B * K)
    tail_comb = jnp.concatenate(
        [input_emb[JT:], output_emb[JT:]], axis=1)
    out = _cbow_sc(target_ids.astype(jnp.int32), ctx_idx, neg_idx,
                   input_emb.T, output_emb.T, tail_comb)
    return (out[:, K], out[:, :K])


# speedup vs baseline: 3.0802x; 2.6178x over previous
import functools

import jax
import jax.numpy as jnp
from jax import lax
from jax.experimental import pallas as pl
from jax.experimental.pallas import tpu as pltpu, tpu_sc as plsc

VOCAB = 1000000
D = 64
B = 16384
C = 20
K = 20

NC = 2
NS = 16
NW = NC * NS

ROWS_W = B // NW
R = 32
NCHUNK = ROWS_W // R
IDX_BLK = 128
NBLK = R * C // IDX_BLK


def _cbow_body(tgt_hbm, ctxidx_hbm, negidx_hbm, in_hbm, out_hbm,
               logits_hbm, idx_ctx_v, idx_neg_v, idx_tgt_v, ctx_rows_v,
               orow_v, out_v, sem):
    wid = lax.axis_index("s") * NC + lax.axis_index("c")
    lane = lax.iota(jnp.int32, 16)

    def chunk_body(ch, carry):
        row0 = wid * ROWS_W + ch * R
        off = row0 * C

        pltpu.sync_copy(ctxidx_hbm.at[pl.ds(off, R * C)], idx_ctx_v)
        pltpu.sync_copy(negidx_hbm.at[pl.ds(off, R * K)], idx_neg_v)
        pltpu.sync_copy(tgt_hbm.at[pl.ds(row0, R)], idx_tgt_v)

        copies = []
        for j in range(NBLK):
            copies.append(pltpu.async_copy(
                in_hbm.at[idx_ctx_v.at[pl.ds(j * IDX_BLK, IDX_BLK)]],
                ctx_rows_v.at[pl.ds(j * IDX_BLK, IDX_BLK)], sem))
            copies.append(pltpu.async_copy(
                out_hbm.at[idx_neg_v.at[pl.ds(j * IDX_BLK, IDX_BLK)]],
                orow_v.at[pl.ds(j * IDX_BLK, IDX_BLK)], sem))
        copies.append(pltpu.async_copy(
            out_hbm.at[idx_tgt_v], orow_v.at[pl.ds(R * K, R)], sem))
        for cp in copies:
            cp.wait()

        def row_body(r, rcarry):
            base = r * C
            acc = [ctx_rows_v[base, pl.ds(j * 16, 16)] for j in range(4)]
            for c in range(1, C):
                for j in range(4):
                    acc[j] = acc[j] + ctx_rows_v[base + c, pl.ds(j * 16, 16)]
            inv_c = jnp.float32(1.0 / C)
            ctxv = [acc[j] * inv_c for j in range(4)]

            def dot(row):
                v = ctxv[0] * orow_v[row, pl.ds(0, 16)]
                for j in range(1, 4):
                    v = v + ctxv[j] * orow_v[row, pl.ds(j * 16, 16)]
                return jnp.sum(v)

            acc1 = jnp.zeros((16,), jnp.float32)
            acc2 = jnp.where(lane == 4, dot(R * K + r),
                             jnp.zeros((16,), jnp.float32))
            for k in range(K):
                s = dot(base + k)
                if k < 16:
                    acc1 = jnp.where(lane == k, s, acc1)
                else:
                    acc2 = jnp.where(lane == k - 16, s, acc2)
            out_v[r, pl.ds(0, 16)] = acc1
            out_v[r, pl.ds(16, 16)] = acc2
            return rcarry

        lax.fori_loop(0, R, row_body, 0)
        pltpu.sync_copy(out_v, logits_hbm.at[pl.ds(row0, R)])
        return carry

    lax.fori_loop(0, NCHUNK, chunk_body, 0)


@functools.partial(jax.jit, static_argnums=())
def _cbow_sc(target_ids, ctx_idx, neg_idx, input_emb, output_emb):
    mesh = plsc.VectorSubcoreMesh(core_axis_name="c", subcore_axis_name="s")
    params = pltpu.CompilerParams(
        needs_layout_passes=False, use_tc_tiling_on_sc=False)
    k = pl.kernel(
        _cbow_body,
        mesh=mesh,
        compiler_params=params,
        out_type=jax.ShapeDtypeStruct((B, 32), jnp.float32),
        scratch_types=[
            pltpu.VMEM((R * C,), jnp.int32),
            pltpu.VMEM((R * K,), jnp.int32),
            pltpu.VMEM((R,), jnp.int32),
            pltpu.VMEM((R * C, D), jnp.float32),
            pltpu.VMEM((R * K + R, D), jnp.float32),
            pltpu.VMEM((R, 32), jnp.float32),
            pltpu.SemaphoreType.DMA,
        ],
    )
    return k(target_ids, ctx_idx, neg_idx, input_emb, output_emb)


def kernel(target_ids, context_ids, negative_ids, input_emb, output_emb):
    ctx_idx = context_ids.astype(jnp.int32).reshape(B * C)
    neg_idx = negative_ids.astype(jnp.int32).reshape(B * K)
    out = _cbow_sc(target_ids.astype(jnp.int32), ctx_idx, neg_idx,
                   input_emb, output_emb)
    return (out[:, K], out[:, :K])


# speedup vs baseline: 3.3759x; 1.0960x over previous
import functools

import jax
import jax.numpy as jnp
from jax import lax
from jax.experimental import pallas as pl
from jax.experimental.pallas import tpu as pltpu, tpu_sc as plsc

VOCAB = 1000000
D = 64
DP = 128
B = 16384
C = 20
K = 20

NC = 2
NS = 16
NW = NC * NS

ROWS_W = B // NW
R = 16
NCHUNK = ROWS_W // R
IDX_BLK = 80
NBLK = R * C // IDX_BLK


def _cbow_body(tgt_hbm, ctxidx_hbm, negidx_hbm, comb_hbm,
               out_hbm, idx_ctx_v, idx_neg_v, idx_tgt_v, ctx_rows_v,
               orow_v, out_v, sem):
    wid = lax.axis_index("s") * NC + lax.axis_index("c")
    lane = lax.iota(jnp.int32, 16)

    def chunk_body(ch, carry):
        row0 = wid * ROWS_W + ch * R
        off = row0 * C

        pltpu.sync_copy(ctxidx_hbm.at[pl.ds(off, R * C)], idx_ctx_v)
        pltpu.sync_copy(negidx_hbm.at[pl.ds(off, R * K)], idx_neg_v)
        pltpu.sync_copy(tgt_hbm.at[pl.ds(row0, R)], idx_tgt_v)

        copies = []
        for j in range(NBLK):
            copies.append(pltpu.async_copy(
                comb_hbm.at[idx_ctx_v.at[pl.ds(j * IDX_BLK, IDX_BLK)]],
                ctx_rows_v.at[pl.ds(j * IDX_BLK, IDX_BLK)], sem))
            copies.append(pltpu.async_copy(
                comb_hbm.at[idx_neg_v.at[pl.ds(j * IDX_BLK, IDX_BLK)]],
                orow_v.at[pl.ds(j * IDX_BLK, IDX_BLK)], sem))
        copies.append(pltpu.async_copy(
            comb_hbm.at[idx_tgt_v], orow_v.at[pl.ds(R * K, R)], sem))
        for cp in copies:
            cp.wait()

        def row_body(r, rcarry):
            base = r * C
            acc = [ctx_rows_v[base, pl.ds(j * 16, 16)] for j in range(4)]
            for c in range(1, C):
                for j in range(4):
                    acc[j] = acc[j] + ctx_rows_v[base + c, pl.ds(j * 16, 16)]
            inv_c = jnp.float32(1.0 / C)
            ctxv = [acc[j] * inv_c for j in range(4)]

            def dot(row):
                v = ctxv[0] * orow_v[row, pl.ds(D, 16)]
                for j in range(1, 4):
                    v = v + ctxv[j] * orow_v[row, pl.ds(D + j * 16, 16)]
                return jnp.sum(v)

            acc1 = jnp.zeros((16,), jnp.float32)
            acc2 = jnp.where(lane == 4, dot(R * K + r),
                             jnp.zeros((16,), jnp.float32))
            for k in range(K):
                s = dot(base + k)
                if k < 16:
                    acc1 = jnp.where(lane == k, s, acc1)
                else:
                    acc2 = jnp.where(lane == k - 16, s, acc2)
            out_v[r, pl.ds(0, 16)] = acc1
            out_v[r, pl.ds(16, 16)] = acc2
            return rcarry

        lax.fori_loop(0, R, row_body, 0)
        pltpu.sync_copy(out_v, out_hbm.at[pl.ds(row0, R)])
        return carry

    lax.fori_loop(0, NCHUNK, chunk_body, 0)


@functools.partial(jax.jit, static_argnums=())
def _cbow_sc(target_ids, ctx_idx, neg_idx, input_emb, output_emb):
    comb = jnp.concatenate([input_emb, output_emb], axis=1)
    mesh = plsc.VectorSubcoreMesh(core_axis_name="c", subcore_axis_name="s")
    params = pltpu.CompilerParams(needs_layout_passes=False)
    k = pl.kernel(
        _cbow_body,
        mesh=mesh,
        compiler_params=params,
        out_type=jax.ShapeDtypeStruct((B, 32), jnp.float32),
        scratch_types=[
            pltpu.VMEM((R * C,), jnp.int32),
            pltpu.VMEM((R * K,), jnp.int32),
            pltpu.VMEM((R,), jnp.int32),
            pltpu.VMEM((R * C, DP), jnp.float32),
            pltpu.VMEM((R * K + R, DP), jnp.float32),
            pltpu.VMEM((R, 32), jnp.float32),
            pltpu.SemaphoreType.DMA,
        ],
    )
    return k(target_ids, ctx_idx, neg_idx, comb)


def kernel(target_ids, context_ids, negative_ids, input_emb, output_emb):
    ctx_idx = context_ids.astype(jnp.int32).reshape(B * C)
    neg_idx = negative_ids.astype(jnp.int32).reshape(B * K)
    out = _cbow_sc(target_ids.astype(jnp.int32), ctx_idx, neg_idx,
                   input_emb, output_emb)
    return (out[:, K], out[:, :K])
